# Initial kernel scaffold; baseline (speedup 1.0000x reference)
import jax
import jax.numpy as jnp
from jax.experimental import pallas as pl


def kernel(feat, lidar2img, bev_table, view_embeds, W_attn, b_attn, W_val, b_val, W_proj, b_proj, W_out, b_out):
    raise NotImplementedError("write your pallas kernel here")


# speedup vs baseline: 51.2769x; 51.2769x over previous
import functools

import jax
import jax.numpy as jnp
from jax import lax
from jax.experimental import pallas as pl
from jax.experimental.pallas import tpu as pltpu
from jax.experimental.pallas import tpu_sc as plsc

V = 6
C = 64
FH = 32
FW = 88
BH = 128
BW = 128
NQ = BH * BW
IMG_H = 512.0
IMG_W = 1408.0
PC = (-51.2, -51.2, -5.0, 51.2, 51.2, 3.0)
EPS = 1e-5
GH = FH + 2
GW = FW + 2
TROWS = GH * GW
NW = 32
RPW = BH // NW


def _val_body(ft_ref, ve_ref, wv_ref, bv_ref, out_ref):
    wvb = wv_ref[...].astype(jnp.bfloat16)
    for v in range(V):
        val = (ft_ref[v] + ve_ref[v]).astype(jnp.bfloat16)
        out_ref[v] = jnp.dot(val, wvb, preferred_element_type=jnp.float32) + bv_ref[...]


def _geom_body(l_ref, idx_ref, wts_ref, cnt_ref):
    colf = lax.broadcasted_iota(jnp.int32, (BH, BW), 1).astype(jnp.float32)
    rowf = lax.broadcasted_iota(jnp.int32, (BH, BW), 0).astype(jnp.float32)
    x = (colf + 0.5) / BW * (PC[3] - PC[0]) + PC[0]
    y = (rowf + 0.5) / BH * (PC[4] - PC[1]) + PC[1]
    z = x * (PC[5] - PC[2]) + PC[2]
    xb = x.astype(jnp.bfloat16).astype(jnp.float32)
    yb = y.astype(jnp.bfloat16).astype(jnp.float32)
    zb = z.astype(jnp.bfloat16).astype(jnp.float32)
    cnt = jnp.zeros((BH, BW), jnp.float32)
    for v in range(V):
        p0 = ((l_ref[v, 0, 0] * xb + l_ref[v, 0, 1] * yb) + l_ref[v, 0, 2] * zb) + l_ref[v, 0, 3]
        p1 = ((l_ref[v, 1, 0] * xb + l_ref[v, 1, 1] * yb) + l_ref[v, 1, 2] * zb) + l_ref[v, 1, 3]
        p2 = ((l_ref[v, 2, 0] * xb + l_ref[v, 2, 1] * yb) + l_ref[v, 2, 2] * zb) + l_ref[v, 2, 3]
        d = jnp.maximum(p2, EPS)
        xn = (p0 / d) / IMG_W
        yn = (p1 / d) / IMG_H
        m = (p2 > EPS) & (xn > 0) & (xn < 1) & (yn > 0) & (yn < 1)
        gx = xn * FW - 0.5
        gy = yn * FH - 0.5
        x0f = jnp.floor(gx)
        y0f = jnp.floor(gy)
        wx = gx - x0f
        wy = gy - y0f
        xg = jnp.where(m, jnp.clip(x0f + 1.0, 0.0, float(FW)), 0.0).astype(jnp.int32)
        yg = jnp.where(m, jnp.clip(y0f + 1.0, 0.0, float(FH)), 0.0).astype(jnp.int32)
        idx_ref[v] = yg * GW + xg + v * TROWS
        zero = jnp.zeros((BH, BW), jnp.float32)
        wts_ref[v, 0] = jnp.where(m, (1.0 - wx) * (1.0 - wy), zero)
        wts_ref[v, 1] = jnp.where(m, wx * (1.0 - wy), zero)
        wts_ref[v, 2] = jnp.where(m, (1.0 - wx) * wy, zero)
        wts_ref[v, 3] = jnp.where(m, wx * wy, zero)
        cnt = cnt + m.astype(jnp.float32)
    cnt_ref[...] = cnt


def _post_body(acc_ref, cnt_ref, bev_ref, wp_ref, bp_ref, wo_ref, bo_ref, out_ref):
    cnt = cnt_ref[...]
    scale = 1.0 / jnp.maximum(cnt, 1.0)
    g = (cnt > 0.0).astype(jnp.float32)
    a = (acc_ref[...] * scale).astype(jnp.bfloat16)
    t = jnp.dot(a, wp_ref[...].astype(jnp.bfloat16), preferred_element_type=jnp.float32) + g * bp_ref[...]
    out_ref[...] = bev_ref[...] + jnp.dot(t.astype(jnp.bfloat16), wo_ref[...].astype(jnp.bfloat16),
                                          preferred_element_type=jnp.float32) + bo_ref[...]


def _sc_body(table_ref, idx_ref, wts_ref, out_ref,
             idx_a, idx_b, wts_a, wts_b, rows_a, rows_b, acc_v, sem_a, sem_b):
    cid = lax.axis_index("c")
    sid = lax.axis_index("s")
    wid = sid * 2 + cid
    base_r = wid * RPW

    steps = [(rr, v) for rr in range(RPW) for v in range(V)]
    nstep = len(steps)
    bufs = [(idx_a, wts_a, rows_a, sem_a), (idx_b, wts_b, rows_b, sem_b)]

    def load_and_issue(t):
        rr, v = steps[t]
        idx_s, wts_s, rows_s, sem_s = bufs[t % 2]
        r = base_r + rr
        pltpu.sync_copy(idx_ref.at[v, r], idx_s)
        for k in range(4):
            pltpu.sync_copy(wts_ref.at[v, k, r], wts_s.at[pl.ds(k * BW, BW)])
        return pltpu.async_copy(table_ref.at[idx_s], rows_s, sem_s)

    handles = [None] * nstep
    handles[0] = load_and_issue(0)
    for t in range(nstep):
        rr, v = steps[t]
        if t + 1 < nstep:
            handles[t + 1] = load_and_issue(t + 1)
        handles[t].wait()
        idx_s, wts_s, rows_s, _ = bufs[t % 2]

        def pbody(p, carry, v=v, wts_s=wts_s, rows_s=rows_s):
            pv = jnp.full((16,), p, jnp.int32)
            w = [plsc.load_gather(wts_s, [pv + (k * BW)]) for k in range(4)]
            for j in range(4):
                s = (w[0] * rows_s[p, pl.ds(j * 16, 16)]
                     + w[1] * rows_s[p, pl.ds(64 + j * 16, 16)]
                     + w[2] * rows_s[p, pl.ds(128 + j * 16, 16)]
                     + w[3] * rows_s[p, pl.ds(192 + j * 16, 16)])
                if v == 0:
                    acc_v[p, pl.ds(j * 16, 16)] = s
                else:
                    acc_v[p, pl.ds(j * 16, 16)] = acc_v[p, pl.ds(j * 16, 16)] + s
            return carry

        lax.fori_loop(0, BW, pbody, None)
        if v == V - 1:
            pltpu.sync_copy(acc_v, out_ref.at[base_r + rr])


def _sc_gather(table, idx, wts):
    mesh = plsc.VectorSubcoreMesh(core_axis_name="c", subcore_axis_name="s")
    fn = pl.kernel(
        _sc_body,
        out_type=jax.ShapeDtypeStruct((BH, BW, C), jnp.float32),
        mesh=mesh,
        compiler_params=pltpu.CompilerParams(needs_layout_passes=False),
        scratch_types=[
            pltpu.VMEM((BW,), jnp.int32),
            pltpu.VMEM((BW,), jnp.int32),
            pltpu.VMEM((4 * BW,), jnp.float32),
            pltpu.VMEM((4 * BW,), jnp.float32),
            pltpu.VMEM((BW, 4 * C), jnp.float32),
            pltpu.VMEM((BW, 4 * C), jnp.float32),
            pltpu.VMEM((BW, C), jnp.float32),
            pltpu.SemaphoreType.DMA,
            pltpu.SemaphoreType.DMA,
        ],
    )
    return fn(table, idx, wts)


def kernel(feat, lidar2img, bev_table, view_embeds, W_attn, b_attn, W_val, b_val, W_proj, b_proj, W_out, b_out):
    featT = feat.transpose(0, 2, 3, 1).reshape(V, FH * FW, C)

    vf = pl.pallas_call(
        _val_body,
        out_shape=jax.ShapeDtypeStruct((V, FH * FW, C), jnp.float32),
    )(featT, view_embeds.reshape(V, 1, C), W_val, b_val.reshape(1, C))

    l_b = lidar2img.astype(jnp.bfloat16).astype(jnp.float32)
    idx, wts, cnt = pl.pallas_call(
        _geom_body,
        in_specs=[pl.BlockSpec(memory_space=pltpu.SMEM)],
        out_shape=[
            jax.ShapeDtypeStruct((V, BH, BW), jnp.int32),
            jax.ShapeDtypeStruct((V, 4, BH, BW), jnp.float32),
            jax.ShapeDtypeStruct((BH, BW), jnp.float32),
        ],
    )(l_b)

    vfg = vf.reshape(V, FH, FW, C)
    vfp = jnp.pad(vfg, ((0, 0), (1, 2), (1, 2), (0, 0)), mode='edge')
    quad = jnp.concatenate(
        [vfp[:, :GH, :GW], vfp[:, :GH, 1:GW + 1],
         vfp[:, 1:GH + 1, :GW], vfp[:, 1:GH + 1, 1:GW + 1]], axis=-1)
    table = quad.reshape(V * TROWS, 4 * C)

    acc = _sc_gather(table, idx, wts).reshape(NQ, C)

    out = pl.pallas_call(
        _post_body,
        grid=(8,),
        in_specs=[
            pl.BlockSpec((NQ // 8, C), lambda i: (i, 0)),
            pl.BlockSpec((NQ // 8, 1), lambda i: (i, 0)),
            pl.BlockSpec((NQ // 8, C), lambda i: (i, 0)),
            pl.BlockSpec((C, C), lambda i: (0, 0)),
            pl.BlockSpec((1, C), lambda i: (0, 0)),
            pl.BlockSpec((C, C), lambda i: (0, 0)),
            pl.BlockSpec((1, C), lambda i: (0, 0)),
        ],
        out_specs=pl.BlockSpec((NQ // 8, C), lambda i: (i, 0)),
        out_shape=jax.ShapeDtypeStruct((NQ, C), jnp.float32),
    )(acc, cnt.reshape(NQ, 1), bev_table, W_proj, b_proj.reshape(1, C), W_out, b_out.reshape(1, C))
    return out


# speedup vs baseline: 52.6283x; 1.0264x over previous
import functools

import jax
import jax.numpy as jnp
from jax import lax
from jax.experimental import pallas as pl
from jax.experimental.pallas import tpu as pltpu
from jax.experimental.pallas import tpu_sc as plsc

V = 6
C = 64
FH = 32
FW = 88
BH = 128
BW = 128
NQ = BH * BW
IMG_H = 512.0
IMG_W = 1408.0
PC = (-51.2, -51.2, -5.0, 51.2, 51.2, 3.0)
EPS = 1e-5
GH = FH + 2
GW = FW + 2
TROWS = GH * GW
NW = 32
RPW = BH // NW


def _val_body(ft_ref, ve_ref, wv_ref, bv_ref, out_ref):
    wvb = wv_ref[...].astype(jnp.bfloat16)
    for v in range(V):
        val = (ft_ref[v] + ve_ref[v]).astype(jnp.bfloat16)
        out_ref[v] = jnp.dot(val, wvb, preferred_element_type=jnp.float32) + bv_ref[...]


def _geom_body(l_ref, idx_ref, wts_ref, cnt_ref):
    colf = lax.broadcasted_iota(jnp.int32, (BH, BW), 1).astype(jnp.float32)
    rowf = lax.broadcasted_iota(jnp.int32, (BH, BW), 0).astype(jnp.float32)
    x = (colf + 0.5) / BW * (PC[3] - PC[0]) + PC[0]
    y = (rowf + 0.5) / BH * (PC[4] - PC[1]) + PC[1]
    z = x * (PC[5] - PC[2]) + PC[2]
    xb = x.astype(jnp.bfloat16).astype(jnp.float32)
    yb = y.astype(jnp.bfloat16).astype(jnp.float32)
    zb = z.astype(jnp.bfloat16).astype(jnp.float32)
    cnt = jnp.zeros((BH, BW), jnp.float32)
    for v in range(V):
        p0 = ((l_ref[v, 0, 0] * xb + l_ref[v, 0, 1] * yb) + l_ref[v, 0, 2] * zb) + l_ref[v, 0, 3]
        p1 = ((l_ref[v, 1, 0] * xb + l_ref[v, 1, 1] * yb) + l_ref[v, 1, 2] * zb) + l_ref[v, 1, 3]
        p2 = ((l_ref[v, 2, 0] * xb + l_ref[v, 2, 1] * yb) + l_ref[v, 2, 2] * zb) + l_ref[v, 2, 3]
        d = jnp.maximum(p2, EPS)
        xn = (p0 / d) / IMG_W
        yn = (p1 / d) / IMG_H
        m = (p2 > EPS) & (xn > 0) & (xn < 1) & (yn > 0) & (yn < 1)
        gx = xn * FW - 0.5
        gy = yn * FH - 0.5
        x0f = jnp.floor(gx)
        y0f = jnp.floor(gy)
        wx = gx - x0f
        wy = gy - y0f
        xg = jnp.where(m, jnp.clip(x0f + 1.0, 0.0, float(FW)), 0.0).astype(jnp.int32)
        yg = jnp.where(m, jnp.clip(y0f + 1.0, 0.0, float(FH)), 0.0).astype(jnp.int32)
        idx_ref[:, pl.ds(v * BW, BW)] = yg * GW + xg + v * TROWS
        zero = jnp.zeros((BH, BW), jnp.float32)
        wts_ref[:, pl.ds((v * 4 + 0) * BW, BW)] = jnp.where(m, (1.0 - wx) * (1.0 - wy), zero)
        wts_ref[:, pl.ds((v * 4 + 1) * BW, BW)] = jnp.where(m, wx * (1.0 - wy), zero)
        wts_ref[:, pl.ds((v * 4 + 2) * BW, BW)] = jnp.where(m, (1.0 - wx) * wy, zero)
        wts_ref[:, pl.ds((v * 4 + 3) * BW, BW)] = jnp.where(m, wx * wy, zero)
        cnt = cnt + m.astype(jnp.float32)
    cnt_ref[...] = cnt


def _post_body(acc_ref, cnt_ref, bev_ref, wp_ref, bp_ref, wo_ref, bo_ref, out_ref):
    cnt = cnt_ref[...]
    scale = 1.0 / jnp.maximum(cnt, 1.0)
    g = (cnt > 0.0).astype(jnp.float32)
    a = (acc_ref[...] * scale).astype(jnp.bfloat16)
    t = jnp.dot(a, wp_ref[...].astype(jnp.bfloat16), preferred_element_type=jnp.float32) + g * bp_ref[...]
    out_ref[...] = bev_ref[...] + jnp.dot(t.astype(jnp.bfloat16), wo_ref[...].astype(jnp.bfloat16),
                                          preferred_element_type=jnp.float32) + bo_ref[...]


def _sc_body(table_ref, idx_ref, wts_ref, out_ref,
             idx_all, wts_all, rows_a, rows_b, acc_v, sem_a, sem_b):
    cid = lax.axis_index("c")
    sid = lax.axis_index("s")
    wid = sid * 2 + cid
    base_r = wid * RPW

    for rr in range(RPW):
        pltpu.sync_copy(idx_ref.at[base_r + rr], idx_all.at[pl.ds(rr * V * BW, V * BW)])
        pltpu.sync_copy(wts_ref.at[base_r + rr], wts_all.at[pl.ds(rr * V * 4 * BW, V * 4 * BW)])

    steps = [(rr, v) for rr in range(RPW) for v in range(V)]
    nstep = len(steps)
    bufs = [(rows_a, sem_a), (rows_b, sem_b)]

    def issue(t):
        rr, v = steps[t]
        rows_s, sem_s = bufs[t % 2]
        return pltpu.async_copy(
            table_ref.at[idx_all.at[pl.ds((rr * V + v) * BW, BW)]], rows_s, sem_s)

    handles = [None] * nstep
    handles[0] = issue(0)
    for t in range(nstep):
        rr, v = steps[t]
        if t + 1 < nstep:
            handles[t + 1] = issue(t + 1)
        handles[t].wait()
        rows_s, _ = bufs[t % 2]
        woff = (rr * V + v) * 4 * BW

        def pbody(p, carry, v=v, woff=woff, rows_s=rows_s):
            pv = jnp.full((16,), p, jnp.int32)
            w = [plsc.load_gather(wts_all, [pv + (woff + k * BW)]) for k in range(4)]
            for j in range(4):
                s = (w[0] * rows_s[p, pl.ds(j * 16, 16)]
                     + w[1] * rows_s[p, pl.ds(64 + j * 16, 16)]
                     + w[2] * rows_s[p, pl.ds(128 + j * 16, 16)]
                     + w[3] * rows_s[p, pl.ds(192 + j * 16, 16)])
                if v == 0:
                    acc_v[p, pl.ds(j * 16, 16)] = s
                else:
                    acc_v[p, pl.ds(j * 16, 16)] = acc_v[p, pl.ds(j * 16, 16)] + s
            return carry

        lax.fori_loop(0, BW, pbody, None, unroll=2)
        if v == V - 1:
            pltpu.sync_copy(acc_v, out_ref.at[base_r + rr])


def _sc_gather(table, idx, wts):
    mesh = plsc.VectorSubcoreMesh(core_axis_name="c", subcore_axis_name="s")
    fn = pl.kernel(
        _sc_body,
        out_type=jax.ShapeDtypeStruct((BH, BW, C), jnp.float32),
        mesh=mesh,
        compiler_params=pltpu.CompilerParams(needs_layout_passes=False),
        scratch_types=[
            pltpu.VMEM((RPW * V * BW,), jnp.int32),
            pltpu.VMEM((RPW * V * 4 * BW,), jnp.float32),
            pltpu.VMEM((BW, 4 * C), jnp.float32),
            pltpu.VMEM((BW, 4 * C), jnp.float32),
            pltpu.VMEM((BW, C), jnp.float32),
            pltpu.SemaphoreType.DMA,
            pltpu.SemaphoreType.DMA,
        ],
    )
    return fn(table, idx, wts)


def kernel(feat, lidar2img, bev_table, view_embeds, W_attn, b_attn, W_val, b_val, W_proj, b_proj, W_out, b_out):
    featT = feat.transpose(0, 2, 3, 1).reshape(V, FH * FW, C)

    vf = pl.pallas_call(
        _val_body,
        out_shape=jax.ShapeDtypeStruct((V, FH * FW, C), jnp.float32),
    )(featT, view_embeds.reshape(V, 1, C), W_val, b_val.reshape(1, C))

    l_b = lidar2img.astype(jnp.bfloat16).astype(jnp.float32)
    idx, wts, cnt = pl.pallas_call(
        _geom_body,
        in_specs=[pl.BlockSpec(memory_space=pltpu.SMEM)],
        out_shape=[
            jax.ShapeDtypeStruct((BH, V * BW), jnp.int32),
            jax.ShapeDtypeStruct((BH, V * 4 * BW), jnp.float32),
            jax.ShapeDtypeStruct((BH, BW), jnp.float32),
        ],
    )(l_b)

    vfg = vf.reshape(V, FH, FW, C)
    vfp = jnp.pad(vfg, ((0, 0), (1, 2), (1, 2), (0, 0)), mode='edge')
    quad = jnp.concatenate(
        [vfp[:, :GH, :GW], vfp[:, :GH, 1:GW + 1],
         vfp[:, 1:GH + 1, :GW], vfp[:, 1:GH + 1, 1:GW + 1]], axis=-1)
    table = quad.reshape(V * TROWS, 4 * C)

    acc = _sc_gather(table, idx, wts).reshape(NQ, C)

    out = pl.pallas_call(
        _post_body,
        grid=(8,),
        in_specs=[
            pl.BlockSpec((NQ // 8, C), lambda i: (i, 0)),
            pl.BlockSpec((NQ // 8, 1), lambda i: (i, 0)),
            pl.BlockSpec((NQ // 8, C), lambda i: (i, 0)),
            pl.BlockSpec((C, C), lambda i: (0, 0)),
            pl.BlockSpec((1, C), lambda i: (0, 0)),
            pl.BlockSpec((C, C), lambda i: (0, 0)),
            pl.BlockSpec((1, C), lambda i: (0, 0)),
        ],
        out_specs=pl.BlockSpec((NQ // 8, C), lambda i: (i, 0)),
        out_shape=jax.ShapeDtypeStruct((NQ, C), jnp.float32),
    )(acc, cnt.reshape(NQ, 1), bev_table, W_proj, b_proj.reshape(1, C), W_out, b_out.reshape(1, C))
    return out


# speedup vs baseline: 55.0251x; 1.0455x over previous
import functools

import jax
import jax.numpy as jnp
from jax import lax
from jax.experimental import pallas as pl
from jax.experimental.pallas import tpu as pltpu
from jax.experimental.pallas import tpu_sc as plsc

V = 6
C = 64
FH = 32
FW = 88
BH = 128
BW = 128
NQ = BH * BW
IMG_H = 512.0
IMG_W = 1408.0
PC = (-51.2, -51.2, -5.0, 51.2, 51.2, 3.0)
EPS = 1e-5
GH = FH + 2
GW = FW + 2
TROWS = GH * GW
NW = 32
RPW = BH // NW


def _val_body(ft_ref, ve_ref, wv_ref, bv_ref, out_ref):
    wvb = wv_ref[...].astype(jnp.bfloat16)
    for v in range(V):
        val = (ft_ref[v] + ve_ref[v]).astype(jnp.bfloat16)
        out_ref[v] = jnp.dot(val, wvb, preferred_element_type=jnp.float32) + bv_ref[...]


def _geom_body(l_ref, idx_ref, wts_ref, cnt_ref):
    colf = lax.broadcasted_iota(jnp.int32, (BH, BW), 1).astype(jnp.float32)
    rowf = lax.broadcasted_iota(jnp.int32, (BH, BW), 0).astype(jnp.float32)
    x = (colf + 0.5) / BW * (PC[3] - PC[0]) + PC[0]
    y = (rowf + 0.5) / BH * (PC[4] - PC[1]) + PC[1]
    z = x * (PC[5] - PC[2]) + PC[2]
    xb = x.astype(jnp.bfloat16).astype(jnp.float32)
    yb = y.astype(jnp.bfloat16).astype(jnp.float32)
    zb = z.astype(jnp.bfloat16).astype(jnp.float32)
    cnt = jnp.zeros((BH, BW), jnp.float32)
    for v in range(V):
        p0 = ((l_ref[v, 0, 0] * xb + l_ref[v, 0, 1] * yb) + l_ref[v, 0, 2] * zb) + l_ref[v, 0, 3]
        p1 = ((l_ref[v, 1, 0] * xb + l_ref[v, 1, 1] * yb) + l_ref[v, 1, 2] * zb) + l_ref[v, 1, 3]
        p2 = ((l_ref[v, 2, 0] * xb + l_ref[v, 2, 1] * yb) + l_ref[v, 2, 2] * zb) + l_ref[v, 2, 3]
        d = jnp.maximum(p2, EPS)
        xn = (p0 / d) / IMG_W
        yn = (p1 / d) / IMG_H
        m = (p2 > EPS) & (xn > 0) & (xn < 1) & (yn > 0) & (yn < 1)
        gx = xn * FW - 0.5
        gy = yn * FH - 0.5
        x0f = jnp.floor(gx)
        y0f = jnp.floor(gy)
        wx = gx - x0f
        wy = gy - y0f
        xg = jnp.where(m, jnp.clip(x0f + 1.0, 0.0, float(FW)), 0.0).astype(jnp.int32)
        yg = jnp.where(m, jnp.clip(y0f + 1.0, 0.0, float(FH)), 0.0).astype(jnp.int32)
        idx_ref[:, pl.ds(v * BW, BW)] = yg * GW + xg + v * TROWS
        zero = jnp.zeros((BH, BW), jnp.float32)
        wts_ref[:, pl.ds((v * 4 + 0) * BW, BW)] = jnp.where(m, (1.0 - wx) * (1.0 - wy), zero)
        wts_ref[:, pl.ds((v * 4 + 1) * BW, BW)] = jnp.where(m, wx * (1.0 - wy), zero)
        wts_ref[:, pl.ds((v * 4 + 2) * BW, BW)] = jnp.where(m, (1.0 - wx) * wy, zero)
        wts_ref[:, pl.ds((v * 4 + 3) * BW, BW)] = jnp.where(m, wx * wy, zero)
        cnt = cnt + m.astype(jnp.float32)
    cnt_ref[...] = cnt


def _post_body(acc_ref, cnt_ref, bev_ref, wp_ref, bp_ref, wo_ref, bo_ref, out_ref):
    cnt = cnt_ref[...]
    scale = 1.0 / jnp.maximum(cnt, 1.0)
    g = (cnt > 0.0).astype(jnp.float32)
    a = (acc_ref[...] * scale).astype(jnp.bfloat16)
    t = jnp.dot(a, wp_ref[...].astype(jnp.bfloat16), preferred_element_type=jnp.float32) + g * bp_ref[...]
    out_ref[...] = bev_ref[...] + jnp.dot(t.astype(jnp.bfloat16), wo_ref[...].astype(jnp.bfloat16),
                                          preferred_element_type=jnp.float32) + bo_ref[...]


def _sc_body(table_ref, idx_ref, wts_ref, out_ref,
             idx_all, wts_all, rows_a, rows_b, acc_v, sem_a, sem_b):
    cid = lax.axis_index("c")
    sid = lax.axis_index("s")
    wid = sid * 2 + cid
    base_r = wid * RPW

    for rr in range(RPW):
        pltpu.sync_copy(idx_ref.at[base_r + rr], idx_all.at[pl.ds(rr * V * BW, V * BW)])
        pltpu.sync_copy(wts_ref.at[base_r + rr], wts_all.at[pl.ds(rr * V * 4 * BW, V * 4 * BW)])

    steps = [(rr, v) for rr in range(RPW) for v in range(V)]
    nstep = len(steps)
    bufs = [(rows_a, sem_a), (rows_b, sem_b)]

    def issue(t):
        rr, v = steps[t]
        rows_s, sem_s = bufs[t % 2]
        return pltpu.async_copy(
            table_ref.at[idx_all.at[pl.ds((rr * V + v) * BW, BW)]], rows_s, sem_s)

    handles = [None] * nstep
    handles[0] = issue(0)
    for t in range(nstep):
        rr, v = steps[t]
        if t + 1 < nstep:
            handles[t + 1] = issue(t + 1)
        handles[t].wait()
        rows_s, _ = bufs[t % 2]
        woff = (rr * V + v) * 4 * BW

        @plsc.parallel_loop(0, BW, unroll=2)
        def pbody(p, v=v, woff=woff, rows_s=rows_s):
            pv = jnp.full((16,), p, jnp.int32)
            w = [plsc.load_gather(wts_all, [pv + (woff + k * BW)]) for k in range(4)]
            for j in range(4):
                s = (w[0] * rows_s[p, pl.ds(j * 16, 16)]
                     + w[1] * rows_s[p, pl.ds(64 + j * 16, 16)]
                     + w[2] * rows_s[p, pl.ds(128 + j * 16, 16)]
                     + w[3] * rows_s[p, pl.ds(192 + j * 16, 16)])
                if v == 0:
                    acc_v[p, pl.ds(j * 16, 16)] = s
                else:
                    acc_v[p, pl.ds(j * 16, 16)] = acc_v[p, pl.ds(j * 16, 16)] + s
        if v == V - 1:
            pltpu.sync_copy(acc_v, out_ref.at[base_r + rr])


def _sc_gather(table, idx, wts):
    mesh = plsc.VectorSubcoreMesh(core_axis_name="c", subcore_axis_name="s")
    fn = pl.kernel(
        _sc_body,
        out_type=jax.ShapeDtypeStruct((BH, BW, C), jnp.float32),
        mesh=mesh,
        compiler_params=pltpu.CompilerParams(needs_layout_passes=False),
        scratch_types=[
            pltpu.VMEM((RPW * V * BW,), jnp.int32),
            pltpu.VMEM((RPW * V * 4 * BW,), jnp.float32),
            pltpu.VMEM((BW, 4 * C), jnp.float32),
            pltpu.VMEM((BW, 4 * C), jnp.float32),
            pltpu.VMEM((BW, C), jnp.float32),
            pltpu.SemaphoreType.DMA,
            pltpu.SemaphoreType.DMA,
        ],
    )
    return fn(table, idx, wts)


def kernel(feat, lidar2img, bev_table, view_embeds, W_attn, b_attn, W_val, b_val, W_proj, b_proj, W_out, b_out):
    featT = feat.transpose(0, 2, 3, 1).reshape(V, FH * FW, C)

    vf = pl.pallas_call(
        _val_body,
        out_shape=jax.ShapeDtypeStruct((V, FH * FW, C), jnp.float32),
    )(featT, view_embeds.reshape(V, 1, C), W_val, b_val.reshape(1, C))

    l_b = lidar2img.astype(jnp.bfloat16).astype(jnp.float32)
    idx, wts, cnt = pl.pallas_call(
        _geom_body,
        in_specs=[pl.BlockSpec(memory_space=pltpu.SMEM)],
        out_shape=[
            jax.ShapeDtypeStruct((BH, V * BW), jnp.int32),
            jax.ShapeDtypeStruct((BH, V * 4 * BW), jnp.float32),
            jax.ShapeDtypeStruct((BH, BW), jnp.float32),
        ],
    )(l_b)

    vfg = vf.reshape(V, FH, FW, C)
    vfp = jnp.pad(vfg, ((0, 0), (1, 2), (1, 2), (0, 0)), mode='edge')
    quad = jnp.concatenate(
        [vfp[:, :GH, :GW], vfp[:, :GH, 1:GW + 1],
         vfp[:, 1:GH + 1, :GW], vfp[:, 1:GH + 1, 1:GW + 1]], axis=-1)
    table = quad.reshape(V * TROWS, 4 * C)

    acc = _sc_gather(table, idx, wts).reshape(NQ, C)

    out = pl.pallas_call(
        _post_body,
        grid=(8,),
        in_specs=[
            pl.BlockSpec((NQ // 8, C), lambda i: (i, 0)),
            pl.BlockSpec((NQ // 8, 1), lambda i: (i, 0)),
            pl.BlockSpec((NQ // 8, C), lambda i: (i, 0)),
            pl.BlockSpec((C, C), lambda i: (0, 0)),
            pl.BlockSpec((1, C), lambda i: (0, 0)),
            pl.BlockSpec((C, C), lambda i: (0, 0)),
            pl.BlockSpec((1, C), lambda i: (0, 0)),
        ],
        out_specs=pl.BlockSpec((NQ // 8, C), lambda i: (i, 0)),
        out_shape=jax.ShapeDtypeStruct((NQ, C), jnp.float32),
    )(acc, cnt.reshape(NQ, 1), bev_table, W_proj, b_proj.reshape(1, C), W_out, b_out.reshape(1, C))
    return out
